# baseline (speedup 1.0000x reference)
import functools

import jax
import jax.numpy as jnp
from jax import lax
from jax.experimental import pallas as pl
from jax.experimental.pallas import tpu as pltpu
from jax.experimental.pallas import tpu_sc as plsc

N_NODES = 256
N_OUT = 64
BATCH = 16384
LANES = 16
NUM_WORKERS = 32
ROWS_PER_WORKER = BATCH // NUM_WORKERS
CHUNK = 128
N_CHUNKS = ROWS_PER_WORKER // CHUNK
N_BLK = CHUNK // LANES
NBUF = 2

_mesh = plsc.VectorSubcoreMesh(core_axis_name="c", subcore_axis_name="s")


@functools.partial(
    pl.kernel,
    mesh=_mesh,
    out_type=jax.ShapeDtypeStruct((N_OUT, BATCH), jnp.float32),
    compiler_params=pltpu.CompilerParams(
        needs_layout_passes=False,
    ),
    scratch_types=[
        pltpu.VMEM((N_OUT,), jnp.int32),
        pltpu.VMEM((N_OUT,), jnp.float32),
        pltpu.VMEM((N_OUT * LANES,), jnp.int32),
        pltpu.VMEM((N_OUT * LANES,), jnp.float32),
        pltpu.VMEM((NBUF, CHUNK, N_NODES), jnp.float32),
        pltpu.VMEM((2, N_OUT, ROWS_PER_WORKER // 2), jnp.float32),
        pltpu.SemaphoreType.DMA,
        pltpu.SemaphoreType.DMA,
    ],
)
def _gather_scale(
    y_hbm, w_hbm, idx_hbm, out_hbm, idx_v, w_v, colb_v, wb_v, in_v, out_v,
    in_sem, out_sem
):
    wid = lax.axis_index("s") * 2 + lax.axis_index("c")
    row0 = wid * ROWS_PER_WORKER

    def start_in(c, buf):
        pltpu.async_copy(
            y_hbm.at[pl.ds(row0 + c * CHUNK, CHUNK)], in_v.at[buf], in_sem
        )

    for c0 in range(NBUF):
        start_in(c0, c0)

    pltpu.sync_copy(idx_hbm, idx_v)
    pltpu.sync_copy(w_hbm, w_v)

    iota = lax.iota(jnp.int32, LANES)
    blk_rows = [lax.broadcast(b * LANES, (LANES,)) + iota for b in range(N_BLK)]

    @plsc.parallel_loop(0, N_OUT, unroll=4)
    def bcast(j):
        j_vec = lax.broadcast(j, (LANES,))
        colb_v[pl.ds(j * LANES, LANES)] = plsc.load_gather(idx_v, [j_vec])
        wb_v[pl.ds(j * LANES, LANES)] = plsc.load_gather(w_v, [j_vec])

    for c in range(N_CHUNKS):
        buf = c % NBUF
        pltpu.make_async_copy(
            y_hbm.at[pl.ds(row0, CHUNK)], in_v.at[buf], in_sem
        ).wait()

        in_c = in_v.at[buf]
        half = c // (N_CHUNKS // 2)
        out_c = out_v.at[half]
        coff = (c % (N_CHUNKS // 2)) * CHUNK

        @plsc.parallel_loop(0, N_OUT, unroll=8)
        def body(j):
            col = colb_v[pl.ds(j * LANES, LANES)]
            wj = wb_v[pl.ds(j * LANES, LANES)]
            for b in range(N_BLK):
                vals = plsc.load_gather(in_c, [blk_rows[b], col])
                out_c[j, pl.ds(coff + b * LANES, LANES)] = vals * wj

        if c % (N_CHUNKS // 2) == N_CHUNKS // 2 - 1:
            pltpu.async_copy(
                out_c,
                out_hbm.at[
                    :, pl.ds(row0 + half * (ROWS_PER_WORKER // 2),
                             ROWS_PER_WORKER // 2)
                ],
                out_sem,
            )
        if c + NBUF < N_CHUNKS:
            start_in(c + NBUF, buf)

    for _ in range(2):
        pltpu.make_async_copy(
            out_v.at[0],
            out_hbm.at[:, pl.ds(row0, ROWS_PER_WORKER // 2)],
            out_sem,
        ).wait()


def kernel(Y_full, weights, output_node_order):
    out_t = _gather_scale(Y_full, weights, output_node_order)
    return out_t.T


# speedup vs baseline: 1.0137x; 1.0137x over previous
import functools

import jax
import jax.numpy as jnp
from jax import lax
from jax.experimental import pallas as pl
from jax.experimental.pallas import tpu as pltpu
from jax.experimental.pallas import tpu_sc as plsc

N_NODES = 256
N_OUT = 64
BATCH = 16384
LANES = 16
NUM_WORKERS = 32
ROWS_PER_WORKER = BATCH // NUM_WORKERS
CHUNK = 128
N_CHUNKS = ROWS_PER_WORKER // CHUNK
N_GROUPS = N_OUT // LANES
NBUF = 2

_mesh = plsc.VectorSubcoreMesh(core_axis_name="c", subcore_axis_name="s")


@functools.partial(
    pl.kernel,
    mesh=_mesh,
    out_type=jax.ShapeDtypeStruct((BATCH, N_OUT), jnp.float32),
    compiler_params=pltpu.CompilerParams(
        needs_layout_passes=False,
    ),
    scratch_types=[
        pltpu.VMEM((N_OUT,), jnp.int32),
        pltpu.VMEM((N_OUT,), jnp.float32),
        pltpu.VMEM((NBUF, CHUNK, N_NODES), jnp.float32),
        pltpu.VMEM((NBUF, CHUNK, N_OUT), jnp.float32),
        pltpu.SemaphoreType.DMA,
        pltpu.SemaphoreType.DMA,
    ],
)
def _gather_scale(
    y_hbm, w_hbm, idx_hbm, out_hbm, idx_v, w_v, in_v, out_v, in_sem, out_sem
):
    wid = lax.axis_index("s") * 2 + lax.axis_index("c")
    row0 = wid * ROWS_PER_WORKER

    pltpu.sync_copy(idx_hbm, idx_v)
    pltpu.sync_copy(w_hbm, w_v)

    idx_vecs = [idx_v[pl.ds(g * LANES, LANES)] for g in range(N_GROUPS)]
    w_vecs = [w_v[pl.ds(g * LANES, LANES)] for g in range(N_GROUPS)]

    def start_in(c, buf):
        pltpu.async_copy(
            y_hbm.at[pl.ds(row0 + c * CHUNK, CHUNK)], in_v.at[buf], in_sem
        )

    start_in(0, 0)
    for c in range(N_CHUNKS):
        buf = c % NBUF
        if c + 1 < N_CHUNKS:
            start_in(c + 1, (c + 1) % NBUF)
        pltpu.make_async_copy(
            y_hbm.at[pl.ds(row0, CHUNK)], in_v.at[buf], in_sem
        ).wait()
        if c >= NBUF:
            pltpu.make_async_copy(
                out_v.at[buf], out_hbm.at[pl.ds(row0, CHUNK)], out_sem
            ).wait()

        in_c = in_v.at[buf]
        out_c = out_v.at[buf]

        @plsc.parallel_loop(0, CHUNK, unroll=4)
        def body(r):
            row_idx = lax.broadcast(r, (LANES,))
            for g in range(N_GROUPS):
                vals = plsc.load_gather(in_c, [row_idx, idx_vecs[g]])
                out_c[r, pl.ds(g * LANES, LANES)] = vals * w_vecs[g]

        pltpu.async_copy(
            out_c, out_hbm.at[pl.ds(row0 + c * CHUNK, CHUNK)], out_sem
        )

    for _ in range(min(NBUF, N_CHUNKS)):
        pltpu.make_async_copy(
            out_v.at[0], out_hbm.at[pl.ds(row0, CHUNK)], out_sem
        ).wait()


def kernel(Y_full, weights, output_node_order):
    return _gather_scale(Y_full, weights, output_node_order)
